# baseline (speedup 1.0000x reference)
import jax
import jax.numpy as jnp
from jax.experimental import pallas as pl
from jax.experimental.pallas import tpu as pltpu

SPLIT_INTO = 7
PERM = (2, 4, 3, 6, 5, 0, 1)
NBUF = 3
GROUP = 8


def kernel(x):
    b, t, f = x.shape
    cwords = (t // SPLIT_INTO) * f
    bwords = t * f
    njobs = b // GROUP

    def body(x_hbm, out_hbm, buf0, buf1, buf2, sem_in, sem_out):
        bufs = (buf0, buf1, buf2)

        def start_in(j):
            return pltpu.make_async_copy(
                x_hbm.at[pl.ds(j * GROUP * bwords, GROUP * bwords)],
                bufs[j % NBUF],
                sem_in.at[j % NBUF],
            )

        def start_outs(j):
            handles = []
            for bi in range(GROUP):
                for c in range(SPLIT_INTO):
                    handles.append(
                        pltpu.make_async_copy(
                            bufs[j % NBUF].at[pl.ds(bi * bwords + PERM[c] * cwords, cwords)],
                            out_hbm.at[
                                pl.ds((j * GROUP + bi) * bwords + c * cwords, cwords)
                            ],
                            sem_out.at[j % NBUF],
                        )
                    )
            for h in handles:
                h.start()
            return handles

        ins, outs = {}, {}
        for j in range(min(NBUF - 1, njobs)):
            ins[j] = start_in(j)
            ins[j].start()
        for j in range(njobs):
            k = j + NBUF - 1
            if k < njobs:
                if k >= NBUF:
                    for h in outs[k - NBUF]:
                        h.wait()
                ins[k] = start_in(k)
                ins[k].start()
            ins[j].wait()
            outs[j] = start_outs(j)
        for j in range(max(0, njobs - NBUF), njobs):
            for h in outs[j]:
                h.wait()

    xflat = x.reshape(b * t * f)
    out = pl.pallas_call(
        body,
        out_shape=jax.ShapeDtypeStruct((b * t * f,), jnp.float32),
        in_specs=[pl.BlockSpec(memory_space=pl.ANY)],
        out_specs=pl.BlockSpec(memory_space=pl.ANY),
        scratch_shapes=[
            pltpu.VMEM((GROUP * bwords,), jnp.float32),
            pltpu.VMEM((GROUP * bwords,), jnp.float32),
            pltpu.VMEM((GROUP * bwords,), jnp.float32),
            pltpu.SemaphoreType.DMA((NBUF,)),
            pltpu.SemaphoreType.DMA((NBUF,)),
        ],
    )(xflat)
    return out.reshape(b, t, f)


# speedup vs baseline: 1.1533x; 1.1533x over previous
import jax
import jax.numpy as jnp
from jax.experimental import pallas as pl
from jax.experimental.pallas import tpu as pltpu

SPLIT_INTO = 7
PERM = (2, 4, 3, 6, 5, 0, 1)
NBUF = 12
AHEAD = 6
SLAB = 4


def kernel(x):
    b, t, f = x.shape
    chunk = t // SPLIT_INTO
    nslab = b // SLAB
    n = SPLIT_INTO * nslab

    def body(x_hbm, out_hbm, buf, sem_in, sem_out):
        def start_in(j):
            c, s = divmod(j, nslab)
            return pltpu.make_async_copy(
                x_hbm.at[pl.ds(s * SLAB, SLAB), pl.ds(PERM[c] * chunk, chunk), :],
                buf.at[j % NBUF],
                sem_in.at[j % NBUF],
            )

        def start_out(j):
            c, s = divmod(j, nslab)
            return pltpu.make_async_copy(
                buf.at[j % NBUF],
                out_hbm.at[pl.ds(s * SLAB, SLAB), pl.ds(c * chunk, chunk), :],
                sem_out.at[j % NBUF],
            )

        ins, outs = {}, {}
        for j in range(AHEAD):
            ins[j] = start_in(j)
            ins[j].start()
        for j in range(n):
            k = j + AHEAD
            if k < n:
                if k >= NBUF:
                    outs[k - NBUF].wait()
                ins[k] = start_in(k)
                ins[k].start()
            ins[j].wait()
            outs[j] = start_out(j)
            outs[j].start()
        for j in range(n - NBUF, n):
            outs[j].wait()

    return pl.pallas_call(
        body,
        out_shape=jax.ShapeDtypeStruct((b, t, f), jnp.float32),
        in_specs=[pl.BlockSpec(memory_space=pl.ANY)],
        out_specs=pl.BlockSpec(memory_space=pl.ANY),
        scratch_shapes=[
            pltpu.VMEM((NBUF, SLAB, chunk, f), jnp.float32),
            pltpu.SemaphoreType.DMA((NBUF,)),
            pltpu.SemaphoreType.DMA((NBUF,)),
        ],
    )(x)


# speedup vs baseline: 4.6155x; 4.0019x over previous
import jax
import jax.numpy as jnp
from jax.experimental import pallas as pl
from jax.experimental.pallas import tpu as pltpu

SPLIT_INTO = 7
PERM = (2, 4, 3, 6, 5, 0, 1)
LANES = 128
NBUF = 4
AHEAD = 2
SLAB = 32


def kernel(x):
    b, t, f = x.shape
    rows = t * f // LANES
    crows = rows // SPLIT_INTO
    nslab = b // SLAB
    n = SPLIT_INTO * nslab

    def body(x_hbm, out_hbm, buf, sem_in, sem_out):
        def start_in(j):
            c, s = divmod(j, nslab)
            return pltpu.make_async_copy(
                x_hbm.at[pl.ds(s * SLAB, SLAB), pl.ds(PERM[c] * crows, crows), :],
                buf.at[j % NBUF],
                sem_in.at[j % NBUF],
            )

        def start_out(j):
            c, s = divmod(j, nslab)
            return pltpu.make_async_copy(
                buf.at[j % NBUF],
                out_hbm.at[pl.ds(s * SLAB, SLAB), pl.ds(c * crows, crows), :],
                sem_out.at[j % NBUF],
            )

        ins, outs = {}, {}
        for j in range(AHEAD):
            ins[j] = start_in(j)
            ins[j].start()
        for j in range(n):
            k = j + AHEAD
            if k < n:
                if k >= NBUF:
                    outs[k - NBUF].wait()
                ins[k] = start_in(k)
                ins[k].start()
            ins[j].wait()
            outs[j] = start_out(j)
            outs[j].start()
        for j in range(n - NBUF, n):
            outs[j].wait()

    xv = x.reshape(b, rows, LANES)
    out = pl.pallas_call(
        body,
        out_shape=jax.ShapeDtypeStruct((b, rows, LANES), jnp.float32),
        in_specs=[pl.BlockSpec(memory_space=pl.ANY)],
        out_specs=pl.BlockSpec(memory_space=pl.ANY),
        scratch_shapes=[
            pltpu.VMEM((NBUF, SLAB, crows, LANES), jnp.float32),
            pltpu.SemaphoreType.DMA((NBUF,)),
            pltpu.SemaphoreType.DMA((NBUF,)),
        ],
    )(xv)
    return out.reshape(b, t, f)
